# baseline (speedup 1.0000x reference)
import functools

import jax
import jax.numpy as jnp
from jax import lax
from jax.experimental import pallas as pl
from jax.experimental.pallas import tpu as pltpu
from jax.experimental.pallas import tpu_sc as plsc

_K = 1024
_C = 256
_N = 8192
_BN = 2048
_GRID = _N // _BN

_NS = 16
_IPW = _N // _NS


def _vq_body(xf_ref, cb_ref, out_ref, idx_ref, loss_ref):
    i = pl.program_id(0)
    xb = xf_ref[...]
    cb = cb_ref[...]
    x_sq = jnp.sum(xb ** 2, axis=-1, keepdims=True)
    cb_sq = jnp.sum(cb ** 2, axis=-1)
    xc = jax.lax.dot_general(xb, cb, (((1,), (1,)), ((), ())))
    d2 = x_sq + cb_sq[None, :] - 2.0 * xc
    m = jnp.min(d2, axis=1, keepdims=True)
    col = jax.lax.broadcasted_iota(jnp.int32, d2.shape, 1)
    idx = jnp.min(jnp.where(d2 == m, col, _K), axis=1)
    idx_ref[...] = idx.reshape(idx_ref.shape)
    onehot = (col == idx[:, None]).astype(jnp.float32)
    q = jax.lax.dot_general(onehot, cb, (((1,), (0,)), ((), ())))
    out_ref[...] = xb + (q - xb)

    @pl.when(i == 0)
    def _init():
        loss_ref[...] = jnp.zeros_like(loss_ref)

    loss_ref[...] += jnp.sum(m).reshape(1, 1)


def _vq_call(xf, codebook):
    return pl.pallas_call(
        _vq_body,
        grid=(_GRID,),
        in_specs=[
            pl.BlockSpec((_BN, _C), lambda i: (i, 0)),
            pl.BlockSpec((_K, _C), lambda i: (0, 0)),
        ],
        out_specs=[
            pl.BlockSpec((_BN, _C), lambda i: (i, 0)),
            pl.BlockSpec((1, 1, _BN), lambda i: (i, 0, 0)),
            pl.BlockSpec((1, 1), lambda i: (0, 0)),
        ],
        out_shape=[
            jax.ShapeDtypeStruct((_N, _C), jnp.float32),
            jax.ShapeDtypeStruct((_GRID, 1, _BN), jnp.int32),
            jax.ShapeDtypeStruct((1, 1), jnp.float32),
        ],
    )(xf, codebook)


def _sc_hist_body(idx_hbm, usage_hbm, idx_v, ones_v, hist_v, out_v, shared):
    cid = lax.axis_index("c")
    sid = lax.axis_index("s")

    @pl.when(cid == 0)
    def _core0():
        pltpu.sync_copy(idx_hbm.at[pl.ds(sid * _IPW, _IPW)], idx_v)

        @pl.when(sid == 0)
        def _zero():
            for k in range(_K // 16):
                hist_v[pl.ds(k * 16, 16)] = jnp.zeros((16,), jnp.int32)
            pltpu.sync_copy(hist_v, shared)

        for k in range(_IPW // 16):
            ones_v[pl.ds(k * 16, 16)] = jnp.ones((16,), jnp.int32)
        plsc.subcore_barrier()
        pltpu.sync_copy(ones_v, shared.at[idx_v], add=True)
        plsc.subcore_barrier()

        @pl.when(sid == 0)
        def _usage():
            pltpu.sync_copy(shared, hist_v)
            one_v = jnp.ones((16,), jnp.int32)
            zero_v = jnp.zeros((16,), jnp.int32)
            acc = zero_v
            for k in range(_K // 16):
                acc = acc + jnp.where(hist_v[pl.ds(k * 16, 16)] == zero_v,
                                      one_v, zero_v)
            out_v[...] = acc.astype(jnp.float32)
            pltpu.sync_copy(out_v, usage_hbm)


def _sc_hist(idx_flat):
    mesh = plsc.VectorSubcoreMesh(core_axis_name="c", subcore_axis_name="s")
    return pl.kernel(
        _sc_hist_body,
        mesh=mesh,
        out_type=jax.ShapeDtypeStruct((16,), jnp.float32),
        scratch_types=[
            pltpu.VMEM((_IPW,), jnp.int32),
            pltpu.VMEM((_IPW,), jnp.int32),
            pltpu.VMEM((_K,), jnp.int32),
            pltpu.VMEM((16,), jnp.float32),
            pltpu.VMEM_SHARED((_K,), jnp.int32),
        ],
    )(idx_flat)


def kernel(x, codebook):
    x = x.astype(jnp.float32)
    B, C, H, W = x.shape
    xf = jnp.transpose(x.reshape(B, C, H * W), (0, 2, 1)).reshape(_N, C)
    q_st, idx3, loss_sum = _vq_call(xf, codebook)
    usage_vec = _sc_hist(idx3.reshape(_N))
    embed_index = idx3.reshape(B, H, W)
    quantize = jnp.transpose(q_st.reshape(B, H * W, C), (0, 2, 1)).reshape(B, C, H, W)
    loss = (loss_sum / float(_N * _C)).reshape(1)
    code_usage = jnp.sum(usage_vec) / _K
    return (quantize, embed_index, loss, code_usage)
